# baseline (speedup 1.0000x reference)
import functools

import jax
import jax.numpy as jnp
from jax import lax
from jax.experimental import pallas as pl
from jax.experimental.pallas import tpu as pltpu
from jax.experimental.pallas import tpu_sc as plsc

_VOCAB = 1_000_000
_D = 64
_L = 200
_B = 4096
_LANES = 16
_NBUF = 4
_CHUNK0 = 128
_CHUNK1 = _L - _CHUNK0


@functools.cache
def _build_kernel():
    info = plsc.get_sparse_core_info()
    nc, ns = info.num_cores, info.num_subcores
    nw = nc * ns
    seq_per_w = _B // nw
    assert _B % nw == 0

    mesh = plsc.VectorSubcoreMesh(core_axis_name="c", subcore_axis_name="s")

    @functools.partial(
        pl.kernel,
        out_type=jax.ShapeDtypeStruct((_B, _L, _D), jnp.float32),
        mesh=mesh,
        scratch_types=[
            pltpu.VMEM((seq_per_w, _L), jnp.int32),
            pltpu.VMEM((_L, _D), jnp.float32),
            [pltpu.VMEM((_L, _D), jnp.float32) for _ in range(_NBUF)],
            [pltpu.SemaphoreType.DMA for _ in range(_NBUF)],
            [pltpu.SemaphoreType.DMA for _ in range(_NBUF)],
        ],
        compiler_params=pltpu.CompilerParams(use_tc_tiling_on_sc=False),
    )
    def emb_kernel(ids_hbm, tok_hbm, pos_hbm, out_hbm, idx_v, pos_v, rows,
                   gsems, ssems):
        wid = lax.axis_index("s") * nc + lax.axis_index("c")
        base = wid * seq_per_w
        pltpu.sync_copy(ids_hbm.at[pl.ds(base, seq_per_w)], idx_v)
        pltpu.sync_copy(pos_hbm, pos_v)

        def start_gather(b, s):
            pltpu.async_copy(
                tok_hbm.at[idx_v.at[s, pl.ds(0, _CHUNK0)]],
                rows[b].at[pl.ds(0, _CHUNK0)], gsems[b])
            pltpu.async_copy(
                tok_hbm.at[idx_v.at[s, pl.ds(_CHUNK0, _CHUNK1)]],
                rows[b].at[pl.ds(_CHUNK0, _CHUNK1)], gsems[b])

        def wait_gather(b):
            pltpu.make_async_copy(tok_hbm.at[pl.ds(0, _L)], rows[b],
                                  gsems[b]).wait()

        def start_scatter(b, s):
            pltpu.async_copy(rows[b], out_hbm.at[base + s], ssems[b])

        def wait_scatter(b):
            pltpu.make_async_copy(rows[b], out_hbm.at[0], ssems[b]).wait()

        def add_pos(b):
            def body(k, carry):
                r0 = k * 8
                for rr in range(8):
                    for c in range(_D // _LANES):
                        v = pos_v[r0 + rr, pl.ds(c * _LANES, _LANES)]
                        plsc.addupdate(
                            rows[b].at[r0 + rr, pl.ds(c * _LANES, _LANES)], v)
                return carry
            lax.fori_loop(0, _L // 8, body, 0)

        start_gather(0, 0)
        start_gather(1, 1)
        for s in range(_NBUF):
            b = s % _NBUF
            wait_gather(b)
            add_pos(b)
            if s >= 2:
                wait_scatter((s + 2) % _NBUF)
            start_gather((s + 2) % _NBUF, s + 2)
            start_scatter(b, s)

        def outer(i, carry):
            s0 = i * _NBUF
            for j in range(_NBUF):
                s = s0 + j
                b = j
                wait_gather(b)
                add_pos(b)
                wait_scatter((j + 2) % _NBUF)
                snext = jnp.minimum(s + 2, seq_per_w - 1)
                start_gather((j + 2) % _NBUF, snext)
                start_scatter(b, s)
            return carry

        lax.fori_loop(1, seq_per_w // _NBUF, outer, 0)
        wait_gather(0)
        wait_gather(1)
        wait_scatter(2)
        wait_scatter(3)

    return emb_kernel


def kernel(input_ids, token_table, position_table):
    ids = input_ids.astype(jnp.int32)
    return _build_kernel()(ids, token_table, position_table)
